# Initial kernel scaffold; baseline (speedup 1.0000x reference)
import jax
import jax.numpy as jnp
from jax.experimental import pallas as pl


def kernel(features, labels, margin, centers):
    raise NotImplementedError("write your pallas kernel here")


# speedup vs baseline: 140.4399x; 140.4399x over previous
import functools

import jax
import jax.numpy as jnp
from jax.experimental import pallas as pl
from jax.experimental.pallas import tpu as pltpu

_C_PAD = 128


def _tcl_kernel(n_classes, feats_ref, labels_ref, centers_ref, margin_ref,
                out_ref):
    f = feats_ref[...]
    c = centers_ref[...]
    labels = labels_ref[...]
    margin = margin_ref[0, 0]

    g = jnp.dot(f, c.T, preferred_element_type=jnp.float32)
    fn = jnp.sum(f * f, axis=1, keepdims=True)
    cn = jnp.sum(c * c, axis=1)[None, :]
    d2 = jnp.maximum(fn - 2.0 * g + cn, 0.0)
    d = jnp.sqrt(d2)

    col = jax.lax.broadcasted_iota(jnp.int32, d.shape, 1)
    pos_mask = col == labels
    valid = col < n_classes

    d_pos = jnp.sum(jnp.where(pos_mask, d, 0.0), axis=1)
    big = jnp.float32(3.0e38)
    d_neg = jnp.min(jnp.where(valid & ~pos_mask, d, big), axis=1)

    per_row = jnp.maximum(d_pos + margin - d_neg, 0.0)
    out_ref[0, 0] = jnp.sum(per_row) / per_row.shape[0]


def kernel(features, labels, margin, centers):
    n_classes, feat_dim = centers.shape
    centers_p = jnp.zeros((_C_PAD, feat_dim), jnp.float32).at[:n_classes].set(
        centers)
    labels2d = labels.reshape(-1, 1)
    margin_arr = jnp.asarray(margin, jnp.float32).reshape(1, 1)

    out = pl.pallas_call(
        functools.partial(_tcl_kernel, n_classes),
        out_shape=jax.ShapeDtypeStruct((1, 1), jnp.float32),
        in_specs=[
            pl.BlockSpec(memory_space=pltpu.VMEM),
            pl.BlockSpec(memory_space=pltpu.VMEM),
            pl.BlockSpec(memory_space=pltpu.VMEM),
            pl.BlockSpec(memory_space=pltpu.SMEM),
        ],
        out_specs=pl.BlockSpec(memory_space=pltpu.SMEM),
    )(features, labels2d, centers_p, margin_arr)
    return out[0, 0]
